# baseline (speedup 1.0000x reference)
import functools

import jax
import jax.numpy as jnp
from jax.experimental import pallas as pl
from jax.experimental.pallas import tpu as pltpu

_VMEM_LIMIT_BYTES = 96 * 1024 * 1024


def _fused_kernel(x_ref, w1_ref, b1_ref, w2_ref, b2_ref, w3_ref, b3_ref,
                  fc1_ref, fc2_ref, o_ref, xp_ref, *, H, W):
    C = x_ref.shape[3]
    HW = H * W
    P = w1_ref.shape[1]

    xb = x_ref[...].reshape(2 * HW, C)
    x16 = xb.astype(jnp.bfloat16)

    xp_ref[...] = jnp.zeros(xp_ref.shape, xp_ref.dtype)
    for i in (0, 1):
        h1 = jnp.dot(x16[i * HW:(i + 1) * HW], w1_ref[...],
                     preferred_element_type=jnp.float32)
        h1 = jnp.maximum(h1 + b1_ref[...], 0.0).astype(jnp.bfloat16)
        xp_ref[1:H + 1, 8:8 + W, i * P:(i + 1) * P] = h1.reshape(H, W, P)

    acc = None
    for kx in range(3):
        xs = xp_ref[:, 7 + kx:7 + kx + W, :]
        cat = jnp.concatenate(
            [xs[ky:ky + H].reshape(HW, 2 * P) for ky in range(3)], axis=1)
        d = jnp.dot(cat, w2_ref[kx], preferred_element_type=jnp.float32)
        acc = d if acc is None else acc + d
    h2p = jnp.maximum(acc + b2_ref[...], 0.0).astype(jnp.bfloat16)
    h2 = jnp.concatenate([h2p[:, :P], h2p[:, P:]], axis=0)

    h3 = jnp.dot(h2, w3_ref[...], preferred_element_type=jnp.float32)
    h3 = h3.reshape(2, HW, C)

    y = jnp.mean(h3, axis=1) + b3_ref[...]
    h = jnp.maximum(jnp.dot(y, fc1_ref[...],
                            preferred_element_type=jnp.float32), 0.0)
    g = jax.nn.sigmoid(jnp.dot(h, fc2_ref[...],
                               preferred_element_type=jnp.float32))

    out = jnp.maximum(h3 * g[:, None, :] + (b3_ref[...] * g)[:, None, :]
                      + xb.reshape(2, HW, C), 0.0)
    o_ref[...] = out.reshape(2, H, W, C)


def kernel(x, w1_oi, w2_oihw, w3_oi, fc1_oi, fc2_oi,
           s1, b1, s2, b2, s3, b3):
    B, C, H, W = x.shape
    P = w1_oi.shape[0]
    Cr = fc1_oi.shape[0]
    f32 = jnp.float32
    bf16 = jnp.bfloat16

    w1t = (w1_oi * s1[:, None]).T.astype(bf16)
    w9 = jnp.transpose(w2_oihw, (2, 3, 1, 0)) * s2[None, None, None, :]
    w2c = jnp.transpose(w9, (1, 0, 2, 3)).reshape(3, 3, P, P)
    eye2 = jnp.eye(2, dtype=w2c.dtype)
    w2p = jnp.einsum('xkio,ab->xkaibo', w2c, eye2)
    w2p = w2p.reshape(3, 6 * P, 2 * P).astype(bf16)
    w3t = (w3_oi * s3[:, None]).T.astype(bf16)
    b2p = jnp.tile(b2.reshape(1, P), (1, 2)).astype(f32)

    x_nhwc = jnp.transpose(x, (0, 2, 3, 1))
    out = pl.pallas_call(
        functools.partial(_fused_kernel, H=H, W=W),
        out_shape=jax.ShapeDtypeStruct((B, H, W, C), f32),
        grid=(B // 2,),
        in_specs=[
            pl.BlockSpec((2, H, W, C), lambda b: (b, 0, 0, 0)),
            pl.BlockSpec((C, P), lambda b: (0, 0)),
            pl.BlockSpec((1, P), lambda b: (0, 0)),
            pl.BlockSpec((3, 6 * P, 2 * P), lambda b: (0, 0, 0)),
            pl.BlockSpec((1, 2 * P), lambda b: (0, 0)),
            pl.BlockSpec((P, C), lambda b: (0, 0)),
            pl.BlockSpec((1, C), lambda b: (0, 0)),
            pl.BlockSpec((C, Cr), lambda b: (0, 0)),
            pl.BlockSpec((Cr, C), lambda b: (0, 0)),
        ],
        out_specs=pl.BlockSpec((2, H, W, C), lambda b: (b, 0, 0, 0)),
        scratch_shapes=[pltpu.VMEM((H + 2, W + 16, 2 * P), bf16)],
        compiler_params=pltpu.CompilerParams(
            dimension_semantics=("parallel",),
            vmem_limit_bytes=_VMEM_LIMIT_BYTES,
        ),
    )(x_nhwc, w1t, b1.reshape(1, P).astype(f32), w2p,
      b2p, w3t, b3.reshape(1, C).astype(f32),
      fc1_oi.T.astype(f32), fc2_oi.T.astype(f32))
    return jnp.transpose(out, (0, 3, 1, 2))


# speedup vs baseline: 1.1498x; 1.1498x over previous
import functools

import jax
import jax.numpy as jnp
from jax.experimental import pallas as pl
from jax.experimental.pallas import tpu as pltpu

_VMEM_LIMIT_BYTES = 96 * 1024 * 1024


def _fused_kernel(x_ref, w1_ref, b1_ref, w2_ref, b2_ref, w3_ref, b3_ref,
                  fc1_ref, fc2_ref, o_ref, xp_ref, *, H, W):
    C = x_ref.shape[3]
    HW = H * W
    P = w1_ref.shape[1]

    xb = x_ref[...].reshape(2 * HW, C)
    x16 = xb.astype(jnp.bfloat16)

    xp_ref[...] = jnp.zeros(xp_ref.shape, xp_ref.dtype)
    for i in (0, 1):
        h1 = jnp.dot(x16[i * HW:(i + 1) * HW], w1_ref[...],
                     preferred_element_type=jnp.float32)
        h1 = jnp.maximum(h1 + b1_ref[...], 0.0).astype(jnp.bfloat16)
        xp_ref[1:H + 1, 8:8 + W, i * P:(i + 1) * P] = h1.reshape(H, W, P)

    acc = None
    for kx in range(3):
        xs = xp_ref[:, 7 + kx:7 + kx + W, :]
        cat = jnp.concatenate(
            [xs[ky:ky + H].reshape(HW, 2 * P) for ky in range(3)], axis=1)
        d = jnp.dot(cat, w2_ref[kx], preferred_element_type=jnp.float32)
        acc = d if acc is None else acc + d
    h2f = jnp.maximum(acc + b2_ref[...], 0.0)
    h2p = h2f.astype(jnp.bfloat16)
    h2 = jnp.concatenate([h2p[:, :P], h2p[:, P:]], axis=0)

    m2 = jnp.mean(h2f, axis=0, keepdims=True)
    m2 = jnp.concatenate([m2[:, :P], m2[:, P:]], axis=0)
    y = jnp.dot(m2.astype(jnp.bfloat16), w3_ref[...],
                preferred_element_type=jnp.float32) + b3_ref[...]
    h = jnp.maximum(jnp.dot(y, fc1_ref[...],
                            preferred_element_type=jnp.float32), 0.0)
    g = jax.nn.sigmoid(jnp.dot(h, fc2_ref[...],
                               preferred_element_type=jnp.float32))

    h3 = jnp.dot(h2, w3_ref[...], preferred_element_type=jnp.float32)
    h3 = h3.reshape(2, HW, C)

    out = jnp.maximum(h3 * g[:, None, :] + (b3_ref[...] * g)[:, None, :]
                      + xb.reshape(2, HW, C), 0.0)
    o_ref[...] = out.reshape(2, H, W, C)


def kernel(x, w1_oi, w2_oihw, w3_oi, fc1_oi, fc2_oi,
           s1, b1, s2, b2, s3, b3):
    B, C, H, W = x.shape
    P = w1_oi.shape[0]
    Cr = fc1_oi.shape[0]
    f32 = jnp.float32
    bf16 = jnp.bfloat16

    w1t = (w1_oi * s1[:, None]).T.astype(bf16)
    w9 = jnp.transpose(w2_oihw, (2, 3, 1, 0)) * s2[None, None, None, :]
    w2c = jnp.transpose(w9, (1, 0, 2, 3)).reshape(3, 3, P, P)
    eye2 = jnp.eye(2, dtype=w2c.dtype)
    w2p = jnp.einsum('xkio,ab->xkaibo', w2c, eye2)
    w2p = w2p.reshape(3, 6 * P, 2 * P).astype(bf16)
    w3t = (w3_oi * s3[:, None]).T.astype(bf16)
    b2p = jnp.tile(b2.reshape(1, P), (1, 2)).astype(f32)

    x_nhwc = jnp.transpose(x, (0, 2, 3, 1))
    out = pl.pallas_call(
        functools.partial(_fused_kernel, H=H, W=W),
        out_shape=jax.ShapeDtypeStruct((B, H, W, C), f32),
        grid=(B // 2,),
        in_specs=[
            pl.BlockSpec((2, H, W, C), lambda b: (b, 0, 0, 0)),
            pl.BlockSpec((C, P), lambda b: (0, 0)),
            pl.BlockSpec((1, P), lambda b: (0, 0)),
            pl.BlockSpec((3, 6 * P, 2 * P), lambda b: (0, 0, 0)),
            pl.BlockSpec((1, 2 * P), lambda b: (0, 0)),
            pl.BlockSpec((P, C), lambda b: (0, 0)),
            pl.BlockSpec((1, C), lambda b: (0, 0)),
            pl.BlockSpec((C, Cr), lambda b: (0, 0)),
            pl.BlockSpec((Cr, C), lambda b: (0, 0)),
        ],
        out_specs=pl.BlockSpec((2, H, W, C), lambda b: (b, 0, 0, 0)),
        scratch_shapes=[pltpu.VMEM((H + 2, W + 16, 2 * P), bf16)],
        compiler_params=pltpu.CompilerParams(
            dimension_semantics=("parallel",),
            vmem_limit_bytes=_VMEM_LIMIT_BYTES,
        ),
    )(x_nhwc, w1t, b1.reshape(1, P).astype(f32), w2p,
      b2p, w3t, b3.reshape(1, C).astype(f32),
      fc1_oi.T.astype(f32), fc2_oi.T.astype(f32))
    return jnp.transpose(out, (0, 3, 1, 2))


# speedup vs baseline: 1.3463x; 1.1709x over previous
import functools

import jax
import jax.numpy as jnp
from jax.experimental import pallas as pl
from jax.experimental.pallas import tpu as pltpu

_VMEM_LIMIT_BYTES = 96 * 1024 * 1024


def _fused_kernel(x_ref, w1_ref, b1_ref, w2_ref, b2_ref, w3_ref, b3_ref,
                  fc1_ref, fc2_ref, o_ref, xp_ref, *, H, W):
    C = x_ref.shape[3]
    HW = H * W
    P = w1_ref.shape[1]

    xb = x_ref[...].reshape(2 * HW, C)
    x16 = xb.astype(jnp.bfloat16)

    xp_ref[...] = jnp.zeros(xp_ref.shape, xp_ref.dtype)
    for i in (0, 1):
        h1 = jnp.dot(x16[i * HW:(i + 1) * HW], w1_ref[...],
                     preferred_element_type=jnp.float32)
        h1 = jnp.maximum(h1 + b1_ref[...], 0.0).astype(jnp.bfloat16)
        xp_ref[1:H + 1, 8:8 + W, i * P:(i + 1) * P] = h1.reshape(H, W, P)

    acc = None
    for kx in range(3):
        xs = xp_ref[:, 7 + kx:7 + kx + W, :]
        cat = jnp.concatenate(
            [xs[ky:ky + H].reshape(HW, 2 * P) for ky in range(3)], axis=1)
        d = jnp.dot(cat, w2_ref[kx], preferred_element_type=jnp.float32)
        acc = d if acc is None else acc + d
    h2f = jnp.maximum(acc + b2_ref[...], 0.0)
    h2p = h2f.astype(jnp.bfloat16)
    h2 = jnp.concatenate([h2p[:, :P], h2p[:, P:]], axis=0)

    m2 = jnp.mean(h2f, axis=0, keepdims=True)
    m2 = jnp.concatenate([m2[:, :P], m2[:, P:]], axis=0)
    y = jnp.dot(m2.astype(jnp.bfloat16), w3_ref[...],
                preferred_element_type=jnp.float32) + b3_ref[...]
    h = jnp.maximum(jnp.dot(y, fc1_ref[...],
                            preferred_element_type=jnp.float32), 0.0)
    g = jax.nn.sigmoid(jnp.dot(h, fc2_ref[...],
                               preferred_element_type=jnp.float32))

    bg = b3_ref[...] * g
    hh = H // 2
    for q in range(4):
        i, r0 = q // 2, (q % 2) * (hh * W)
        rows = slice(i * HW + r0, i * HW + r0 + hh * W)
        h3q = jnp.dot(h2[rows], w3_ref[...],
                      preferred_element_type=jnp.float32)
        outq = jnp.maximum(h3q * g[i:i + 1] + bg[i:i + 1] + xb[rows], 0.0)
        o_ref[i, (q % 2) * hh:(q % 2) * hh + hh, :, :] = (
            outq.reshape(hh, W, C))


def kernel(x, w1_oi, w2_oihw, w3_oi, fc1_oi, fc2_oi,
           s1, b1, s2, b2, s3, b3):
    B, C, H, W = x.shape
    P = w1_oi.shape[0]
    Cr = fc1_oi.shape[0]
    f32 = jnp.float32
    bf16 = jnp.bfloat16

    w1t = (w1_oi * s1[:, None]).T.astype(bf16)
    w9 = jnp.transpose(w2_oihw, (2, 3, 1, 0)) * s2[None, None, None, :]
    w2c = jnp.transpose(w9, (1, 0, 2, 3)).reshape(3, 3, P, P)
    eye2 = jnp.eye(2, dtype=w2c.dtype)
    w2p = jnp.einsum('xkio,ab->xkaibo', w2c, eye2)
    w2p = w2p.reshape(3, 6 * P, 2 * P).astype(bf16)
    w3t = (w3_oi * s3[:, None]).T.astype(bf16)
    b2p = jnp.tile(b2.reshape(1, P), (1, 2)).astype(f32)

    x_nhwc = jnp.transpose(x, (0, 2, 3, 1))
    out = pl.pallas_call(
        functools.partial(_fused_kernel, H=H, W=W),
        out_shape=jax.ShapeDtypeStruct((B, H, W, C), f32),
        grid=(B // 2,),
        in_specs=[
            pl.BlockSpec((2, H, W, C), lambda b: (b, 0, 0, 0)),
            pl.BlockSpec((C, P), lambda b: (0, 0)),
            pl.BlockSpec((1, P), lambda b: (0, 0)),
            pl.BlockSpec((3, 6 * P, 2 * P), lambda b: (0, 0, 0)),
            pl.BlockSpec((1, 2 * P), lambda b: (0, 0)),
            pl.BlockSpec((P, C), lambda b: (0, 0)),
            pl.BlockSpec((1, C), lambda b: (0, 0)),
            pl.BlockSpec((C, Cr), lambda b: (0, 0)),
            pl.BlockSpec((Cr, C), lambda b: (0, 0)),
        ],
        out_specs=pl.BlockSpec((2, H, W, C), lambda b: (b, 0, 0, 0)),
        scratch_shapes=[pltpu.VMEM((H + 2, W + 16, 2 * P), bf16)],
        compiler_params=pltpu.CompilerParams(
            dimension_semantics=("parallel",),
            vmem_limit_bytes=_VMEM_LIMIT_BYTES,
        ),
    )(x_nhwc, w1t, b1.reshape(1, P).astype(f32), w2p,
      b2p, w3t, b3.reshape(1, C).astype(f32),
      fc1_oi.T.astype(f32), fc2_oi.T.astype(f32))
    return jnp.transpose(out, (0, 3, 1, 2))


# speedup vs baseline: 1.3928x; 1.0345x over previous
import functools

import jax
import jax.numpy as jnp
from jax.experimental import pallas as pl
from jax.experimental.pallas import tpu as pltpu

_VMEM_LIMIT_BYTES = 96 * 1024 * 1024


def _fused_kernel(x_ref, w1_ref, b1_ref, w2_ref, b2_ref, w3_ref, b3_ref,
                  fc1_ref, fc2_ref, o_ref, xp_ref, *, H, W):
    C = x_ref.shape[3]
    HW = H * W
    P = w1_ref.shape[1]

    xb = x_ref[...].reshape(2 * HW, C)
    x16 = xb.astype(jnp.bfloat16)

    xp_ref[...] = jnp.zeros(xp_ref.shape, xp_ref.dtype)
    for i in (0, 1):
        h1 = jnp.dot(x16[i * HW:(i + 1) * HW], w1_ref[...],
                     preferred_element_type=jnp.float32)
        h1 = jnp.maximum(h1 + b1_ref[...], 0.0).astype(jnp.bfloat16)
        xp_ref[1:H + 1, 8:8 + W, i * P:(i + 1) * P] = h1.reshape(H, W, P)

    hh2 = H // 2
    halves = [None, None]
    for kx in range(3):
        xs = xp_ref[:, 7 + kx:7 + kx + W, :]
        for rh in range(2):
            base = rh * hh2
            cat = jnp.concatenate(
                [xs[base + ky:base + ky + hh2].reshape(hh2 * W, 2 * P)
                 for ky in range(3)], axis=1)
            d = jnp.dot(cat, w2_ref[kx], preferred_element_type=jnp.float32)
            halves[rh] = d if halves[rh] is None else halves[rh] + d
    acc = jnp.concatenate(halves, axis=0)
    h2f = jnp.maximum(acc + b2_ref[...], 0.0)
    h2p = h2f.astype(jnp.bfloat16)
    h2 = jnp.concatenate([h2p[:, :P], h2p[:, P:]], axis=0)

    m2 = jnp.mean(h2f, axis=0, keepdims=True)
    m2 = jnp.concatenate([m2[:, :P], m2[:, P:]], axis=0)
    y = jnp.dot(m2.astype(jnp.bfloat16), w3_ref[...],
                preferred_element_type=jnp.float32) + b3_ref[...]
    h = jnp.maximum(jnp.dot(y, fc1_ref[...],
                            preferred_element_type=jnp.float32), 0.0)
    g = jax.nn.sigmoid(jnp.dot(h, fc2_ref[...],
                               preferred_element_type=jnp.float32))

    bg = b3_ref[...] * g
    hh = H // 2
    for q in range(4):
        i, r0 = q // 2, (q % 2) * (hh * W)
        rows = slice(i * HW + r0, i * HW + r0 + hh * W)
        h3q = jnp.dot(h2[rows], w3_ref[...],
                      preferred_element_type=jnp.float32)
        outq = jnp.maximum(h3q * g[i:i + 1] + bg[i:i + 1] + xb[rows], 0.0)
        o_ref[i, (q % 2) * hh:(q % 2) * hh + hh, :, :] = (
            outq.reshape(hh, W, C))


def kernel(x, w1_oi, w2_oihw, w3_oi, fc1_oi, fc2_oi,
           s1, b1, s2, b2, s3, b3):
    B, C, H, W = x.shape
    P = w1_oi.shape[0]
    Cr = fc1_oi.shape[0]
    f32 = jnp.float32
    bf16 = jnp.bfloat16

    w1t = (w1_oi * s1[:, None]).T.astype(bf16)
    w9 = jnp.transpose(w2_oihw, (2, 3, 1, 0)) * s2[None, None, None, :]
    w2c = jnp.transpose(w9, (1, 0, 2, 3)).reshape(3, 3, P, P)
    eye2 = jnp.eye(2, dtype=w2c.dtype)
    w2p = jnp.einsum('xkio,ab->xkaibo', w2c, eye2)
    w2p = w2p.reshape(3, 6 * P, 2 * P).astype(bf16)
    w3t = (w3_oi * s3[:, None]).T.astype(bf16)
    b2p = jnp.tile(b2.reshape(1, P), (1, 2)).astype(f32)

    x_nhwc = jnp.transpose(x, (0, 2, 3, 1))
    out = pl.pallas_call(
        functools.partial(_fused_kernel, H=H, W=W),
        out_shape=jax.ShapeDtypeStruct((B, H, W, C), f32),
        grid=(B // 2,),
        in_specs=[
            pl.BlockSpec((2, H, W, C), lambda b: (b, 0, 0, 0)),
            pl.BlockSpec((C, P), lambda b: (0, 0)),
            pl.BlockSpec((1, P), lambda b: (0, 0)),
            pl.BlockSpec((3, 6 * P, 2 * P), lambda b: (0, 0, 0)),
            pl.BlockSpec((1, 2 * P), lambda b: (0, 0)),
            pl.BlockSpec((P, C), lambda b: (0, 0)),
            pl.BlockSpec((1, C), lambda b: (0, 0)),
            pl.BlockSpec((C, Cr), lambda b: (0, 0)),
            pl.BlockSpec((Cr, C), lambda b: (0, 0)),
        ],
        out_specs=pl.BlockSpec((2, H, W, C), lambda b: (b, 0, 0, 0)),
        scratch_shapes=[pltpu.VMEM((H + 2, W + 16, 2 * P), bf16)],
        compiler_params=pltpu.CompilerParams(
            dimension_semantics=("parallel",),
            vmem_limit_bytes=_VMEM_LIMIT_BYTES,
        ),
    )(x_nhwc, w1t, b1.reshape(1, P).astype(f32), w2p,
      b2p, w3t, b3.reshape(1, C).astype(f32),
      fc1_oi.T.astype(f32), fc2_oi.T.astype(f32))
    return jnp.transpose(out, (0, 3, 1, 2))
